# Initial kernel scaffold; baseline (speedup 1.0000x reference)
import functools

import jax
import jax.numpy as jnp
from jax import lax
from jax.experimental import pallas as pl
from jax.experimental.pallas import tpu as pltpu
from jax.experimental.pallas import tpu_sc as plsc

NC = 2
NS = 16
NW = NC * NS

CH = 125
D = 128
SW = 16


def _sc_segment_sums(table, srcs, dsts, sides, E, rows_pad):
    n_chunks = E // (NW * CH)
    epw = n_chunks * CH
    rpt = rows_pad // NS
    assert rows_pad % (NS * 128) == 0

    mesh = plsc.VectorSubcoreMesh(core_axis_name="c", subcore_axis_name="s")

    @functools.partial(
        pl.kernel,
        out_type=[
            jax.ShapeDtypeStruct((3, NC, rows_pad, D), jnp.float32),
            jax.ShapeDtypeStruct((3, NC, rows_pad, SW), jnp.float32),
        ],
        mesh=mesh,
        scratch_types=[
            pltpu.VMEM_SHARED((rows_pad, D), jnp.float32),
            pltpu.VMEM_SHARED((rows_pad, SW), jnp.float32),
            pltpu.VMEM((n_chunks, CH), jnp.int32),
            pltpu.VMEM((n_chunks, CH), jnp.int32),
            pltpu.VMEM((CH, D), jnp.float32),
            pltpu.VMEM((CH, SW), jnp.float32),
            pltpu.VMEM((128, D), jnp.float32),
            pltpu.VMEM((128, SW), jnp.float32),
            pltpu.SemaphoreType.DMA,
        ],
    )
    def sc_seg(table_h, s0, d0, e0, s1, d1, e1, s2, d2, e2, pF, pS,
               accF, accS, src_v, dst_v, rows_v, side_v, zF, zS, sem):
        c = lax.axis_index("c")
        s = lax.axis_index("s")
        w = c * NS + s

        zvec = jnp.zeros((16,), jnp.float32)

        def zero_row(i, carry):
            for j in range(D // 16):
                zF[i, pl.ds(j * 16, 16)] = zvec
            zS[i, pl.ds(0, 16)] = zvec
            return carry

        lax.fori_loop(0, 128, zero_row, 0)

        for t, (src_h, dst_h, side_h) in enumerate(
                ((s0, d0, e0), (s1, d1, e1), (s2, d2, e2))):
            for k in range(rpt // 128):
                pltpu.sync_copy(zF, accF.at[pl.ds(s * rpt + k * 128, 128)])
                pltpu.sync_copy(zS, accS.at[pl.ds(s * rpt + k * 128, 128)])
            plsc.subcore_barrier()

            pltpu.sync_copy(src_h.at[pl.ds(w * n_chunks, n_chunks)], src_v)
            pltpu.sync_copy(dst_h.at[pl.ds(w * n_chunks, n_chunks)], dst_v)

            def chunk(j, carry):
                pltpu.async_copy(table_h.at[src_v.at[j]], rows_v, sem).wait()
                pltpu.sync_copy(side_h.at[pl.ds(w * epw + j * CH, CH)], side_v)
                pltpu.sync_copy(rows_v, accF.at[dst_v.at[j]], add=True)
                pltpu.sync_copy(side_v, accS.at[dst_v.at[j]], add=True)
                return carry

            lax.fori_loop(0, n_chunks, chunk, 0)
            plsc.subcore_barrier()

            pltpu.sync_copy(accF.at[pl.ds(s * rpt, rpt)],
                            pF.at[t, c, pl.ds(s * rpt, rpt)])
            pltpu.sync_copy(accS.at[pl.ds(s * rpt, rpt)],
                            pS.at[t, c, pl.ds(s * rpt, rpt)])

    return sc_seg(table, srcs[0], dsts[0], sides[0], srcs[1], dsts[1],
                  sides[1], srcs[2], dsts[2], sides[2])


def _tc_combine(pF, pS, Wa, Wb, instr, fin, blk=400):
    n = instr.shape[0]
    grid = (n // blk,)

    def body(pf_ref, ps_ref, wa_ref, wb_ref, in_ref, fi_ref, oi_ref, of_ref):
        means = []
        for t in range(3):
            accF = pf_ref[t, 0] + pf_ref[t, 1]
            accS = ps_ref[t, 0] + ps_ref[t, 1]
            cnt = accS[:, 4:5]
            sums = (jnp.dot(accF, wa_ref[t], preferred_element_type=jnp.float32)
                    + jnp.dot(accS[:, :8], wb_ref[t],
                              preferred_element_type=jnp.float32))
            means.append(sums / jnp.maximum(cnt, 1.0))
        oi_ref[...] = jnp.maximum(in_ref[...] + 0.5 * (means[0] + means[1]), 0.0)
        of_ref[...] = jnp.maximum(fi_ref[...] + means[2], 0.0)

    return pl.pallas_call(
        body,
        grid=grid,
        in_specs=[
            pl.BlockSpec((3, NC, blk, D), lambda i: (0, 0, i, 0)),
            pl.BlockSpec((3, NC, blk, SW), lambda i: (0, 0, i, 0)),
            pl.BlockSpec((3, D, D), lambda i: (0, 0, 0)),
            pl.BlockSpec((3, 8, D), lambda i: (0, 0, 0)),
            pl.BlockSpec((blk, D), lambda i: (i, 0)),
            pl.BlockSpec((blk, D), lambda i: (i, 0)),
        ],
        out_specs=[
            pl.BlockSpec((blk, D), lambda i: (i, 0)),
            pl.BlockSpec((blk, D), lambda i: (i, 0)),
        ],
        out_shape=[
            jax.ShapeDtypeStruct((n, D), jnp.float32),
            jax.ShapeDtypeStruct((n, D), jnp.float32),
        ],
    )(pF, pS, Wa, Wb, instr, fin)


def kernel(instruction_feats, final_feats, edge_index_prev, edge_index_succ,
           edge_index_final, edge_feats_prev, edge_feats_succ, edge_feats_final,
           W_prev, b_prev, W_succ, b_succ, W_final, b_final):
    E = edge_index_prev.shape[1]
    n = instruction_feats.shape[0]
    rows_pad = ((n + NS * 128 - 1) // (NS * 128)) * (NS * 128)

    srcs, dsts, sides = [], [], []
    for eidx, ef in ((edge_index_prev, edge_feats_prev),
                     (edge_index_succ, edge_feats_succ),
                     (edge_index_final, edge_feats_final)):
        srcs.append(eidx[0].astype(jnp.int32).reshape(E // CH, CH))
        dsts.append(eidx[1].astype(jnp.int32).reshape(E // CH, CH))
        sides.append(jnp.concatenate(
            [ef.astype(jnp.float32),
             jnp.ones((E, 1), jnp.float32),
             jnp.zeros((E, SW - ef.shape[1] - 1), jnp.float32)], axis=1))

    pF, pS = _sc_segment_sums(instruction_feats, srcs, dsts, sides, E, rows_pad)

    de = edge_feats_prev.shape[1]
    Wa = jnp.stack([W_prev[:D], W_succ[:D], W_final[:D]])
    Wb = jnp.stack([
        jnp.concatenate([W[D:], b[None, :], jnp.zeros((8 - de - 1, D), jnp.float32)])
        for W, b in ((W_prev, b_prev), (W_succ, b_succ), (W_final, b_final))])

    pF = pF[:, :, :n]
    pS = pS[:, :, :n]
    return _tc_combine(pF, pS, Wa, Wb, instruction_feats, final_feats)


# speedup vs baseline: 1.5736x; 1.5736x over previous
import functools

import jax
import jax.numpy as jnp
from jax import lax
from jax.experimental import pallas as pl
from jax.experimental.pallas import tpu as pltpu
from jax.experimental.pallas import tpu_sc as plsc

NC = 2
NS = 16
NW = NC * NS

CH = 128
D = 128
IDX_BLK = 16
HALF = 5056
ACC_ROWS = 5120
JUNK = ACC_ROWS - 1
SF = 5


def _sc_segment_sums(table, srcs, dsts, sides, E):
    n_chunks = E // (NW * CH)
    rpt = ACC_ROWS // NS
    assert n_chunks % IDX_BLK == 0 and ACC_ROWS % (NS * 8) == 0

    zf = jnp.zeros((rpt, D), jnp.float32)
    mesh = plsc.VectorSubcoreMesh(core_axis_name="c", subcore_axis_name="s")

    @functools.partial(
        pl.kernel,
        out_type=[
            jax.ShapeDtypeStruct((3, NC, ACC_ROWS, D), jnp.float32),
            jax.ShapeDtypeStruct((3, NC, ACC_ROWS, D), jnp.float32),
            jax.ShapeDtypeStruct((3, NC, ACC_ROWS, D), jnp.float32),
            jax.ShapeDtypeStruct((3, NC, ACC_ROWS, D), jnp.float32),
        ],
        mesh=mesh,
        scratch_types=[
            pltpu.VMEM_SHARED((ACC_ROWS, D), jnp.float32),
            pltpu.VMEM_SHARED((ACC_ROWS, D), jnp.float32),
            pltpu.VMEM((IDX_BLK, CH), jnp.int32),
            pltpu.VMEM((IDX_BLK, CH), jnp.int32),
            pltpu.VMEM((1, CH), jnp.int32),
            pltpu.VMEM((1, CH), jnp.int32),
            pltpu.VMEM((CH, D), jnp.float32),
            pltpu.VMEM((CH, 16), jnp.float32),
            pltpu.SemaphoreType.DMA,
        ],
    )
    def sc_seg(table_h, zf_h, s0, d0, e0, s1, d1, e1, s2, d2, e2,
               pFA, pFB, pSA, pSB,
               accA, accB, src_v, dst_v, idxA, idxB, rows_v, side_v, sem):
        c = lax.axis_index("c")
        s = lax.axis_index("s")
        w = c * NS + s

        def route(jj):
            for v in range(CH // 16):
                dvec = dst_v[jj, pl.ds(v * 16, 16)]
                in_a = dvec < HALF
                idxA[0, pl.ds(v * 16, 16)] = jnp.where(in_a, dvec, JUNK)
                idxB[0, pl.ds(v * 16, 16)] = jnp.where(
                    in_a, JUNK, dvec - HALF)

        def zero_accs():
            pltpu.sync_copy(zf_h, accA.at[pl.ds(s * rpt, rpt)])
            pltpu.sync_copy(zf_h, accB.at[pl.ds(s * rpt, rpt)])
            plsc.subcore_barrier()

        def dump_accs(outA, outB, t):
            sl = pl.ds(s * rpt, rpt)
            pltpu.sync_copy(accA.at[sl], outA.at[t, c, sl])
            pltpu.sync_copy(accB.at[sl], outB.at[t, c, sl])

        for t, (src_h, dst_h, side_h) in enumerate(
                ((s0, d0, e0), (s1, d1, e1), (s2, d2, e2))):
            zero_accs()

            def blk(ib, carry):
                base = w * n_chunks + ib * IDX_BLK
                pltpu.sync_copy(src_h.at[pl.ds(base, IDX_BLK)], src_v)
                pltpu.sync_copy(dst_h.at[pl.ds(base, IDX_BLK)], dst_v)

                for jj in range(IDX_BLK):
                    gcp = pltpu.async_copy(table_h.at[src_v.at[jj]], rows_v,
                                           sem)
                    route(jj)
                    gcp.wait()
                    pltpu.sync_copy(rows_v, accA.at[idxA.at[0]], add=True)
                    pltpu.sync_copy(rows_v, accB.at[idxB.at[0]], add=True)
                return carry

            lax.fori_loop(0, n_chunks // IDX_BLK, blk, 0)
            plsc.subcore_barrier()
            dump_accs(pFA, pFB, t)
            plsc.subcore_barrier()

            zero_accs()

            def sblk(ib, carry):
                base = w * n_chunks + ib * IDX_BLK
                pltpu.sync_copy(dst_h.at[pl.ds(base, IDX_BLK)], dst_v)

                for jj in range(IDX_BLK):
                    pltpu.sync_copy(side_h.at[base + jj], side_v)

                    def srow(r, carry2):
                        rows_v[r, pl.ds(0, 16)] = side_v[r, pl.ds(0, 16)]
                        return carry2

                    lax.fori_loop(0, CH, srow, 0)
                    route(jj)
                    pltpu.sync_copy(rows_v, accA.at[idxA.at[0]], add=True)
                    pltpu.sync_copy(rows_v, accB.at[idxB.at[0]], add=True)
                return carry

            lax.fori_loop(0, n_chunks // IDX_BLK, sblk, 0)
            plsc.subcore_barrier()
            dump_accs(pSA, pSB, t)
            plsc.subcore_barrier()

    return sc_seg(table, zf, srcs[0], dsts[0], sides[0], srcs[1], dsts[1],
                  sides[1], srcs[2], dsts[2], sides[2])


def _tc_combine(pF, pS, Wa, Wb, instr, fin, blk=400):
    n = instr.shape[0]
    grid = (n // blk,)

    def body(pf_ref, ps_ref, wa_ref, wb_ref, in_ref, fi_ref, oi_ref, of_ref):
        means = []
        for t in range(3):
            accF = pf_ref[t, 0] + pf_ref[t, 1]
            accS = ps_ref[t, 0] + ps_ref[t, 1]
            cnt = accS[:, 4:5]
            sums = (jnp.dot(accF, wa_ref[t], preferred_element_type=jnp.float32)
                    + jnp.dot(accS, wb_ref[t],
                              preferred_element_type=jnp.float32))
            means.append(sums / jnp.maximum(cnt, 1.0))
        oi_ref[...] = jnp.maximum(in_ref[...] + 0.5 * (means[0] + means[1]), 0.0)
        of_ref[...] = jnp.maximum(fi_ref[...] + means[2], 0.0)

    return pl.pallas_call(
        body,
        grid=grid,
        in_specs=[
            pl.BlockSpec((3, NC, blk, D), lambda i: (0, 0, i, 0)),
            pl.BlockSpec((3, NC, blk, 8), lambda i: (0, 0, i, 0)),
            pl.BlockSpec((3, D, D), lambda i: (0, 0, 0)),
            pl.BlockSpec((3, 8, D), lambda i: (0, 0, 0)),
            pl.BlockSpec((blk, D), lambda i: (i, 0)),
            pl.BlockSpec((blk, D), lambda i: (i, 0)),
        ],
        out_specs=[
            pl.BlockSpec((blk, D), lambda i: (i, 0)),
            pl.BlockSpec((blk, D), lambda i: (i, 0)),
        ],
        out_shape=[
            jax.ShapeDtypeStruct((n, D), jnp.float32),
            jax.ShapeDtypeStruct((n, D), jnp.float32),
        ],
    )(pF, pS, Wa, Wb, instr, fin)


def kernel(instruction_feats, final_feats, edge_index_prev, edge_index_succ,
           edge_index_final, edge_feats_prev, edge_feats_succ, edge_feats_final,
           W_prev, b_prev, W_succ, b_succ, W_final, b_final):
    E = edge_index_prev.shape[1]
    n = instruction_feats.shape[0]

    quantum = NW * CH * IDX_BLK
    E_pad = ((E + quantum - 1) // quantum) * quantum
    pad = E_pad - E

    srcs, dsts, sides = [], [], []
    for eidx, ef in ((edge_index_prev, edge_feats_prev),
                     (edge_index_succ, edge_feats_succ),
                     (edge_index_final, edge_feats_final)):
        src = jnp.concatenate(
            [eidx[0].astype(jnp.int32), jnp.zeros((pad,), jnp.int32)])
        dst = jnp.concatenate(
            [eidx[1].astype(jnp.int32), jnp.full((pad,), n, jnp.int32)])
        srcs.append(src.reshape(E_pad // CH, CH))
        dsts.append(dst.reshape(E_pad // CH, CH))
        side = jnp.concatenate(
            [ef.astype(jnp.float32),
             jnp.ones((E, 1), jnp.float32),
             jnp.zeros((E, 16 - ef.shape[1] - 1), jnp.float32)], axis=1)
        side = jnp.concatenate([side, jnp.zeros((pad, 16), jnp.float32)])
        sides.append(side.reshape(E_pad // CH, CH, 16))

    pFA, pFB, pSA, pSB = _sc_segment_sums(
        instruction_feats, srcs, dsts, sides, E_pad)

    pF = jnp.concatenate([pFA[:, :, :HALF], pFB[:, :, :n - HALF]], axis=2)
    pS = jnp.concatenate(
        [pSA[:, :, :HALF, :8], pSB[:, :, :n - HALF, :8]], axis=2)

    de = edge_feats_prev.shape[1]
    Wa = jnp.stack([W_prev[:D], W_succ[:D], W_final[:D]])
    Wb = jnp.stack([
        jnp.concatenate([W[D:], b[None, :], jnp.zeros((8 - de - 1, D), jnp.float32)])
        for W, b in ((W_prev, b_prev), (W_succ, b_succ), (W_final, b_final))])

    return _tc_combine(pF, pS, Wa, Wb, instruction_feats, final_feats)
